# baseline (speedup 1.0000x reference)
import functools

import jax
import jax.numpy as jnp
from jax import lax
from jax.experimental import pallas as pl
from jax.experimental.pallas import tpu as pltpu
from jax.experimental.pallas import tpu_sc as plsc

B = 4096
D = 128
L = 50
LP = 56
NC, NS = 2, 16
NW = NC * NS
BPW = B // NW
IDS_PW = BPW * LP
CROWS = 2
CIDX = CROWS * LP
NCHUNK = BPW // CROWS
DC = D // 16

BM = 512

_sc_mesh = plsc.VectorSubcoreMesh(
    core_axis_name="c", subcore_axis_name="s", num_cores=NC, num_subcores=NS)


@functools.partial(
    pl.kernel,
    out_type=[
        jax.ShapeDtypeStruct((B, D), jnp.float32),
        jax.ShapeDtypeStruct((B, D), jnp.float32),
        jax.ShapeDtypeStruct((B,), jnp.float32),
    ],
    mesh=_sc_mesh,
    scratch_types=[
        pltpu.VMEM((BPW, 128), jnp.int32),
        pltpu.VMEM((BPW, 128), jnp.float32),
        pltpu.VMEM((BPW,), jnp.int32),
        pltpu.VMEM((CIDX, D), jnp.float32),
        pltpu.VMEM((CIDX, D), jnp.float32),
        pltpu.VMEM((BPW, D), jnp.float32),
        pltpu.VMEM((BPW, D), jnp.float32),
        pltpu.VMEM((IDS_PW + 8,), jnp.float32),
        pltpu.VMEM((BPW,), jnp.float32),
        pltpu.VMEM((BPW,), jnp.float32),
        pltpu.SemaphoreType.DMA,
        pltpu.SemaphoreType.DMA,
        pltpu.SemaphoreType.DMA,
    ],
)
def _sc_gather(pkid_hbm, pkrat_hbm, iid_hbm, emb_hbm, lin_hbm,
               user_hbm, item_hbm, linsum_hbm,
               pkid_v, pkrat_v, iid_v, rows0, rows1, user_v,
               itemrows_v, linv_v, itemlin_v, linsum_v, sem0, sem1, sem2):
    wid = lax.axis_index("c") * NS + lax.axis_index("s")
    bbase = wid * BPW

    pltpu.sync_copy(pkid_hbm.at[pl.ds(bbase, BPW)], pkid_v)
    pltpu.sync_copy(pkrat_hbm.at[pl.ds(bbase, BPW)], pkrat_v)
    pltpu.sync_copy(iid_hbm.at[pl.ds(bbase, BPW)], iid_v)

    pltpu.make_async_copy(emb_hbm.at[iid_v], itemrows_v, sem2).start()
    pltpu.make_async_copy(lin_hbm.at[iid_v], itemlin_v, sem2).start()

    rows_bufs = (rows0, rows1)
    sems = (sem0, sem1)

    def start_chunk(c, buf):
        for r in range(CROWS):
            idx = pkid_v.at[c * CROWS + r, pl.ds(0, L)]
            pltpu.make_async_copy(
                emb_hbm.at[idx], rows_bufs[buf].at[pl.ds(r * LP, L)],
                sems[buf]).start()
            pltpu.make_async_copy(
                lin_hbm.at[idx],
                linv_v.at[pl.ds((c * CROWS + r) * LP, L)], sems[buf]).start()

    def wait_chunk(c, buf):
        for r in range(CROWS):
            idx = pkid_v.at[c * CROWS + r, pl.ds(0, L)]
            pltpu.make_async_copy(
                emb_hbm.at[idx], rows_bufs[buf].at[pl.ds(r * LP, L)],
                sems[buf]).wait()
            pltpu.make_async_copy(
                lin_hbm.at[idx],
                linv_v.at[pl.ds((c * CROWS + r) * LP, L)], sems[buf]).wait()

    _dnums = lax.GatherDimensionNumbers(
        offset_dims=(), collapsed_slice_dims=(0,), start_index_map=(0,))

    def _bcast(vec, lane):
        return lax.gather(vec, jnp.full((16, 1), lane, jnp.int32), _dnums, (1,),
                          mode=lax.GatherScatterMode.PROMISE_IN_BOUNDS)

    def _lanesum(v):
        for sh in (8, 4, 2, 1):
            perm = (jnp.arange(16, dtype=jnp.int32) ^ sh).reshape(16, 1)
            v = v + lax.gather(v, perm, _dnums, (1,),
                               mode=lax.GatherScatterMode.PROMISE_IN_BOUNDS)
        return v

    _mask2 = jnp.arange(16, dtype=jnp.int32) < 2
    _lane = jnp.arange(16, dtype=jnp.int32)
    _zero16 = jnp.zeros((16,), jnp.float32)
    RC = 0

    def compute_chunk(c, buf):
        rows = rows_bufs[buf]
        lin_sums = []
        for r in range(CROWS):
            brow = c * CROWS + r
            lbase = brow * LP

            def group(g, acc, _r=r, _brow=brow, _lbase=lbase):
                *emb_acc, lacc = acc
                rg = pkrat_v[_brow, pl.ds(RC + g * 16, 16)]
                lacc = lacc + rg * linv_v[pl.ds(_lbase + g * 16, 16)]
                row0 = _r * LP + g * 16
                for li in range(16):
                    rv = _bcast(rg, li)
                    emb_acc = [emb_acc[dc]
                               + rv * rows[row0 + li, pl.ds(dc * 16, 16)]
                               for dc in range(DC)]
                return (*emb_acc, lacc)

            init = tuple(_zero16 for _ in range(DC + 1))
            *accs, lacc = lax.fori_loop(0, 3, group, init)
            rg = pkrat_v[brow, pl.ds(RC + 48, 16)]
            lacc = lacc + jnp.where(_mask2,
                                    rg * linv_v[pl.ds(lbase + 48, 16)], 0.0)
            for li in range(L - 48):
                rv = _bcast(rg, li)
                accs = [accs[dc] + rv * rows[r * LP + 48 + li,
                                             pl.ds(dc * 16, 16)]
                        for dc in range(DC)]
            for dc in range(DC):
                user_v[brow, pl.ds(dc * 16, 16)] = accs[dc]
            lin_sums.append(_lanesum(lacc))
        return lin_sums

    start_chunk(0, 0)
    start_chunk(1, 1)

    def insert_lin(lvec, sums, row0):
        base = row0 % 16
        for k, s in enumerate(sums):
            lvec = jnp.where(_lane == base + k, s, lvec)
        return lvec

    def main_body(i, lvec):
        c0 = 2 * i
        wait_chunk(c0, 0)
        s01 = compute_chunk(c0, 0)
        start_chunk(c0 + 2, 0)
        wait_chunk(c0 + 1, 1)
        s23 = compute_chunk(c0 + 1, 1)
        start_chunk(c0 + 3, 1)
        lvec = insert_lin(lvec, s01 + s23, 4 * i)
        linsum_v[pl.ds(((4 * i) // 16) * 16, 16)] = lvec
        return lvec

    lvec = lax.fori_loop(0, NCHUNK // 2 - 1, main_body, _zero16)
    wait_chunk(NCHUNK - 2, 0)
    sa = compute_chunk(NCHUNK - 2, 0)
    wait_chunk(NCHUNK - 1, 1)
    sb = compute_chunk(NCHUNK - 1, 1)
    lvec = insert_lin(lvec, sa + sb, BPW - 4)
    linsum_v[pl.ds(BPW - 16, 16)] = lvec

    pltpu.sync_copy(user_v, user_hbm.at[pl.ds(bbase, BPW)])
    pltpu.make_async_copy(emb_hbm.at[iid_v], itemrows_v, sem2).wait()
    pltpu.make_async_copy(lin_hbm.at[iid_v], itemlin_v, sem2).wait()
    pltpu.sync_copy(itemrows_v, item_hbm.at[pl.ds(bbase, BPW)])
    for g in range(BPW // 16):
        linsum_v[pl.ds(g * 16, 16)] = (linsum_v[pl.ds(g * 16, 16)]
                                       + itemlin_v[pl.ds(g * 16, 16)])
    pltpu.sync_copy(linsum_v, linsum_hbm.at[pl.ds(bbase, BPW)])


def _mlp_body(u_ref, i_ref, lin_ref,
              w0_ref, b0_ref, w1_ref, b1_ref, w2_ref, b2_ref, w3_ref, c0_ref,
              out_ref):
    u = u_ref[...]
    it = i_ref[...]
    x = jnp.concatenate([u, it], axis=1)
    h = jnp.maximum(jnp.dot(x, w0_ref[...],
                            preferred_element_type=jnp.float32) + b0_ref[...], 0.0)
    h = jnp.maximum(jnp.dot(h, w1_ref[...],
                            preferred_element_type=jnp.float32) + b1_ref[...], 0.0)
    h = jnp.maximum(jnp.dot(h, w2_ref[...],
                            preferred_element_type=jnp.float32) + b2_ref[...], 0.0)
    mlp = jnp.sum(h * w3_ref[...], axis=1, keepdims=True)
    fm = jnp.sum(u * it, axis=1, keepdims=True)
    out_ref[...] = jax.nn.sigmoid(lin_ref[...] + fm + mlp + c0_ref[...])


def _mlp_call(user_emb, item_emb, linsum, W0, b0, W1, b1, W2, b2, w3_row, c0):
    grid = (B // BM,)
    full = lambda shape: pl.BlockSpec(shape, lambda i: (0,) * len(shape))
    return pl.pallas_call(
        _mlp_body,
        grid=grid,
        in_specs=[
            pl.BlockSpec((BM, D), lambda i: (i, 0)),
            pl.BlockSpec((BM, D), lambda i: (i, 0)),
            pl.BlockSpec((BM, 1), lambda i: (i, 0)),
            full(W0.shape), full(b0.shape),
            full(W1.shape), full(b1.shape),
            full(W2.shape), full(b2.shape),
            full(w3_row.shape), full(c0.shape),
        ],
        out_specs=pl.BlockSpec((BM, 1), lambda i: (i, 0)),
        out_shape=jax.ShapeDtypeStruct((B, 1), jnp.float32),
    )(user_emb, item_emb, linsum,
      W0, b0, W1, b1, W2, b2, w3_row, c0)


def kernel(feature_ids, feature_ratings, item_ids, emb_table, lin_table, lin_bias,
           W0, b0, W1, b1, W2, b2, W3, b3):
    fid = feature_ids.astype(jnp.int32)
    iid = item_ids.astype(jnp.int32)

    pk_ids = jnp.concatenate(
        [fid, fid[:, :LP - L], jnp.zeros((B, 128 - LP), jnp.int32)], axis=1)
    pk_rat = jnp.concatenate(
        [feature_ratings, jnp.zeros((B, 128 - L), jnp.float32)], axis=1)

    user_emb, item_emb, linsum = _sc_gather(
        pk_ids, pk_rat, iid, emb_table, lin_table.reshape(-1))

    c0 = (b3 + lin_bias).reshape(1, 1)
    out = _mlp_call(user_emb, item_emb, linsum.reshape(B, 1),
                    W0, b0.reshape(1, -1), W1, b1.reshape(1, -1),
                    W2, b2.reshape(1, -1), W3.reshape(1, -1), c0)
    return out[:, 0]


# speedup vs baseline: 1.1661x; 1.1661x over previous
import functools

import jax
import jax.numpy as jnp
from jax import lax
from jax.experimental import pallas as pl
from jax.experimental.pallas import tpu as pltpu
from jax.experimental.pallas import tpu_sc as plsc

B = 4096
D = 128
L = 50
LP = 56
NC, NS = 2, 16
NW = NC * NS
BPW = B // NW
IDS_PW = BPW * LP
CROWS = 2
CIDX = CROWS * LP
NCHUNK = BPW // CROWS
DC = D // 16

BM = 512

_sc_mesh = plsc.VectorSubcoreMesh(
    core_axis_name="c", subcore_axis_name="s", num_cores=NC, num_subcores=NS)


@functools.partial(
    pl.kernel,
    out_type=[
        jax.ShapeDtypeStruct((B, D), jnp.float32),
        jax.ShapeDtypeStruct((B, D), jnp.float32),
        jax.ShapeDtypeStruct((B,), jnp.float32),
    ],
    mesh=_sc_mesh,
    scratch_types=[
        pltpu.VMEM((IDS_PW,), jnp.int32),
        pltpu.VMEM((IDS_PW + 8,), jnp.float32),
        pltpu.VMEM((BPW,), jnp.int32),
        pltpu.VMEM((CIDX, D), jnp.float32),
        pltpu.VMEM((CIDX, D), jnp.float32),
        pltpu.VMEM((BPW, D), jnp.float32),
        pltpu.VMEM((BPW, D), jnp.float32),
        pltpu.VMEM((IDS_PW + 8,), jnp.float32),
        pltpu.VMEM((BPW,), jnp.float32),
        pltpu.VMEM((BPW,), jnp.float32),
        pltpu.SemaphoreType.DMA,
        pltpu.SemaphoreType.DMA,
        pltpu.SemaphoreType.DMA,
    ],
)
def _sc_gather(fid_hbm, rat_hbm, iid_hbm, emb_hbm, lin_hbm,
               user_hbm, item_hbm, linsum_hbm,
               fid_v, rat_v, iid_v, rows0, rows1, user_v,
               itemrows_v, linv_v, itemlin_v, linsum_v, sem0, sem1, sem2):
    wid = lax.axis_index("c") * NS + lax.axis_index("s")
    ibase = wid * IDS_PW
    bbase = wid * BPW

    pltpu.sync_copy(fid_hbm.at[pl.ds(ibase, IDS_PW)], fid_v)
    pltpu.sync_copy(rat_hbm.at[pl.ds(ibase, IDS_PW)], rat_v.at[pl.ds(0, IDS_PW)])
    pltpu.sync_copy(iid_hbm.at[pl.ds(bbase, BPW)], iid_v)

    pltpu.make_async_copy(emb_hbm.at[iid_v], itemrows_v, sem2).start()
    pltpu.make_async_copy(lin_hbm.at[iid_v], itemlin_v, sem2).start()

    rows_bufs = (rows0, rows1)
    sems = (sem0, sem1)

    def start_chunk(c, buf):
        for r in range(CROWS):
            idx = fid_v.at[pl.ds((c * CROWS + r) * LP, L)]
            pltpu.make_async_copy(
                emb_hbm.at[idx], rows_bufs[buf].at[pl.ds(r * LP, L)],
                sems[buf]).start()
            pltpu.make_async_copy(
                lin_hbm.at[idx],
                linv_v.at[pl.ds((c * CROWS + r) * LP, L)], sems[buf]).start()

    def wait_chunk(c, buf):
        for r in range(CROWS):
            idx = fid_v.at[pl.ds((c * CROWS + r) * LP, L)]
            pltpu.make_async_copy(
                emb_hbm.at[idx], rows_bufs[buf].at[pl.ds(r * LP, L)],
                sems[buf]).wait()
            pltpu.make_async_copy(
                lin_hbm.at[idx],
                linv_v.at[pl.ds((c * CROWS + r) * LP, L)], sems[buf]).wait()

    _dnums = lax.GatherDimensionNumbers(
        offset_dims=(), collapsed_slice_dims=(0,), start_index_map=(0,))

    def _bcast(vec, lane):
        return lax.gather(vec, jnp.full((16, 1), lane, jnp.int32), _dnums, (1,),
                          mode=lax.GatherScatterMode.PROMISE_IN_BOUNDS)

    def _lanesum(v):
        for sh in (8, 4, 2, 1):
            perm = (jnp.arange(16, dtype=jnp.int32) ^ sh).reshape(16, 1)
            v = v + lax.gather(v, perm, _dnums, (1,),
                               mode=lax.GatherScatterMode.PROMISE_IN_BOUNDS)
        return v

    _mask2 = jnp.arange(16, dtype=jnp.int32) < 2
    _lane = jnp.arange(16, dtype=jnp.int32)
    _zero16 = jnp.zeros((16,), jnp.float32)

    def compute_chunk(c, buf):
        rows = rows_bufs[buf]
        lin_sums = []
        for r in range(CROWS):
            brow = c * CROWS + r
            lbase = brow * LP

            def group(g, acc, _r=r, _brow=brow, _lbase=lbase):
                *emb_acc, lacc = acc
                rg = rat_v[pl.ds(_lbase + g * 16, 16)]
                lacc = lacc + rg * linv_v[pl.ds(_lbase + g * 16, 16)]
                row0 = _r * LP + g * 16
                for li in range(16):
                    rv = _bcast(rg, li)
                    emb_acc = [emb_acc[dc]
                               + rv * rows[row0 + li, pl.ds(dc * 16, 16)]
                               for dc in range(DC)]
                return (*emb_acc, lacc)

            init = tuple(_zero16 for _ in range(DC + 1))
            *accs, lacc = lax.fori_loop(0, 3, group, init)
            rg = rat_v[pl.ds(lbase + 48, 16)]
            lacc = lacc + jnp.where(_mask2,
                                    rg * linv_v[pl.ds(lbase + 48, 16)], 0.0)
            for li in range(L - 48):
                rv = _bcast(rg, li)
                accs = [accs[dc] + rv * rows[r * LP + 48 + li,
                                             pl.ds(dc * 16, 16)]
                        for dc in range(DC)]
            for dc in range(DC):
                user_v[brow, pl.ds(dc * 16, 16)] = accs[dc]
            lin_sums.append(_lanesum(lacc))
        return lin_sums

    start_chunk(0, 0)
    start_chunk(1, 1)

    def insert_lin(lvec, sums, row0):
        base = row0 % 16
        for k, s in enumerate(sums):
            lvec = jnp.where(_lane == base + k, s, lvec)
        return lvec

    def main_body(i, lvec):
        c0 = 2 * i
        wait_chunk(c0, 0)
        s01 = compute_chunk(c0, 0)
        start_chunk(c0 + 2, 0)
        wait_chunk(c0 + 1, 1)
        s23 = compute_chunk(c0 + 1, 1)
        start_chunk(c0 + 3, 1)
        lvec = insert_lin(lvec, s01 + s23, 4 * i)
        linsum_v[pl.ds(((4 * i) // 16) * 16, 16)] = lvec
        return lvec

    lvec = lax.fori_loop(0, NCHUNK // 2 - 1, main_body, _zero16)
    wait_chunk(NCHUNK - 2, 0)
    sa = compute_chunk(NCHUNK - 2, 0)
    wait_chunk(NCHUNK - 1, 1)
    sb = compute_chunk(NCHUNK - 1, 1)
    lvec = insert_lin(lvec, sa + sb, BPW - 4)
    linsum_v[pl.ds(BPW - 16, 16)] = lvec

    pltpu.sync_copy(user_v, user_hbm.at[pl.ds(bbase, BPW)])
    pltpu.make_async_copy(emb_hbm.at[iid_v], itemrows_v, sem2).wait()
    pltpu.make_async_copy(lin_hbm.at[iid_v], itemlin_v, sem2).wait()
    pltpu.sync_copy(itemrows_v, item_hbm.at[pl.ds(bbase, BPW)])
    for g in range(BPW // 16):
        linsum_v[pl.ds(g * 16, 16)] = (linsum_v[pl.ds(g * 16, 16)]
                                       + itemlin_v[pl.ds(g * 16, 16)])
    pltpu.sync_copy(linsum_v, linsum_hbm.at[pl.ds(bbase, BPW)])


def _mlp_body(u_ref, i_ref, lin_ref,
              w0_ref, b0_ref, w1_ref, b1_ref, w2_ref, b2_ref, w3_ref, c0_ref,
              out_ref):
    u = u_ref[...]
    it = i_ref[...]
    x = jnp.concatenate([u, it], axis=1).astype(jnp.bfloat16)
    h = jnp.maximum(jnp.dot(x, w0_ref[...],
                            preferred_element_type=jnp.float32) + b0_ref[...],
                    0.0).astype(jnp.bfloat16)
    h = jnp.maximum(jnp.dot(h, w1_ref[...],
                            preferred_element_type=jnp.float32) + b1_ref[...],
                    0.0).astype(jnp.bfloat16)
    h = jnp.maximum(jnp.dot(h, w2_ref[...],
                            preferred_element_type=jnp.float32) + b2_ref[...], 0.0)
    mlp = jnp.sum(h * w3_ref[...], axis=1, keepdims=True)
    fm = jnp.sum(u * it, axis=1, keepdims=True)
    out_ref[...] = jax.nn.sigmoid(lin_ref[...] + fm + mlp + c0_ref[...])


def _mlp_call(user_emb, item_emb, linsum, W0, b0, W1, b1, W2, b2, w3_row, c0):
    grid = (B // BM,)
    full = lambda shape: pl.BlockSpec(shape, lambda i: (0,) * len(shape))
    return pl.pallas_call(
        _mlp_body,
        grid=grid,
        in_specs=[
            pl.BlockSpec((BM, D), lambda i: (i, 0)),
            pl.BlockSpec((BM, D), lambda i: (i, 0)),
            pl.BlockSpec((BM, 1), lambda i: (i, 0)),
            full(W0.shape), full(b0.shape),
            full(W1.shape), full(b1.shape),
            full(W2.shape), full(b2.shape),
            full(w3_row.shape), full(c0.shape),
        ],
        out_specs=pl.BlockSpec((BM, 1), lambda i: (i, 0)),
        out_shape=jax.ShapeDtypeStruct((B, 1), jnp.float32),
    )(user_emb, item_emb, linsum,
      W0, b0, W1, b1, W2, b2, w3_row, c0)


def kernel(feature_ids, feature_ratings, item_ids, emb_table, lin_table, lin_bias,
           W0, b0, W1, b1, W2, b2, W3, b3):
    fid = feature_ids.astype(jnp.int32)
    iid = item_ids.astype(jnp.int32)

    fid_p = jnp.concatenate([fid, fid[:, :LP - L]], axis=1)
    rat_p = jnp.pad(feature_ratings, ((0, 0), (0, LP - L)))

    user_emb, item_emb, linsum = _sc_gather(
        fid_p.reshape(-1), rat_p.reshape(-1), iid,
        emb_table, lin_table.reshape(-1))

    c0 = (b3 + lin_bias).reshape(1, 1)
    out = _mlp_call(user_emb, item_emb, linsum.reshape(B, 1),
                    W0.astype(jnp.bfloat16), b0.reshape(1, -1),
                    W1.astype(jnp.bfloat16), b1.reshape(1, -1),
                    W2.astype(jnp.bfloat16), b2.reshape(1, -1),
                    W3.reshape(1, -1), c0)
    return out[:, 0]
